# baseline (speedup 1.0000x reference)
import jax
import jax.numpy as jnp
from jax import lax
from jax.experimental import pallas as pl
from jax.experimental.pallas import tpu as pltpu, tpu_sc as plsc

T = 4096
D = 128
LANES = 16
NC = 2
NS = 16
NW = NC * NS
CHUNK = T // NW
CHUNK_E = CHUNK * D
KD = D // LANES
GROUP = 16
TAIL_W = D + LANES

_mesh = plsc.VectorSubcoreMesh(core_axis_name="c", subcore_axis_name="s")
_cparams = pltpu.CompilerParams(needs_layout_passes=False)


def _cu_scalars(cu_vec):
    lane = lax.iota(jnp.int32, LANES)
    cu_f = cu_vec.astype(jnp.float32)
    c1 = jnp.sum(jnp.where(lane == 1, cu_f, 0.0)).astype(jnp.int32)
    c2 = jnp.sum(jnp.where(lane == 2, cu_f, 0.0)).astype(jnp.int32)
    c3 = jnp.sum(jnp.where(lane == 3, cu_f, 0.0)).astype(jnp.int32)
    return c1, c2, c3


def _seg_of(p, c1, c2, c3):
    return ((p >= c1).astype(jnp.int32) + (p >= c2).astype(jnp.int32)
            + (p >= c3).astype(jnp.int32))


def _row_update(ctx_v, th, off, rg, b1, b2, b3, den, nums, keep_out=None):
    cks = [ctx_v[pl.ds(off + LANES * k, LANES)] for k in range(KD)]
    acc = cks[0] * th[0]
    for k in range(1, KD):
        acc = acc + cks[k] * th[k]
    e = jnp.exp(jnp.full((LANES,), jnp.sum(acc), jnp.float32))
    is_start = (rg == b1) | (rg == b2) | (rg == b3)
    kv = jnp.full((LANES,), jnp.where(is_start, 0.0, 1.0), jnp.float32)
    den = den * kv + e
    nums = tuple(n * kv + e * ck for n, ck in zip(nums, cks))
    return den, nums, cks, e


def _l1_body(ctx_hbm, cu_hbm, th_hbm, tails_hbm, ctx_v, th_v, cu_v, tl_v):
    c = lax.axis_index("c")
    s = lax.axis_index("s")
    w = s * NC + c
    pltpu.sync_copy(ctx_hbm.at[pl.ds(w * CHUNK_E, CHUNK_E)], ctx_v)
    pltpu.sync_copy(th_hbm, th_v)
    pltpu.sync_copy(cu_hbm, cu_v)
    c1, c2, c3 = _cu_scalars(cu_v[:])
    th = [th_v[pl.ds(LANES * k, LANES)] for k in range(KD)]
    row0 = w * CHUNK
    b1, b2, b3 = c1 - row0, c2 - row0, c3 - row0
    zero = jnp.zeros((LANES,), jnp.float32)

    def p(g, carry):
        den = carry[0]
        nums = carry[1:]
        for j in range(GROUP):
            rg = g * GROUP + j
            den, nums, _, _ = _row_update(ctx_v, th, rg * D, rg,
                                          b1, b2, b3, den, nums)
        return (den,) + nums

    res = lax.fori_loop(0, CHUNK // GROUP, p, (zero,) * (KD + 1))
    for k in range(KD):
        tl_v[pl.ds(LANES * k, LANES)] = res[1 + k]
    tl_v[pl.ds(D, LANES)] = res[0]
    pltpu.sync_copy(tl_v, tails_hbm.at[pl.ds(w * TAIL_W, TAIL_W)])


def _l2_body(ctx_hbm, cu_hbm, th_hbm, tails_hbm, out_hbm,
             ctx_v, th_v, cu_v, ta_v):
    c = lax.axis_index("c")
    s = lax.axis_index("s")
    w = s * NC + c
    pltpu.sync_copy(ctx_hbm.at[pl.ds(w * CHUNK_E, CHUNK_E)], ctx_v)
    pltpu.sync_copy(th_hbm, th_v)
    pltpu.sync_copy(cu_hbm, cu_v)
    pltpu.sync_copy(tails_hbm, ta_v)
    c1, c2, c3 = _cu_scalars(cu_v[:])
    th = [th_v[pl.ds(LANES * k, LANES)] for k in range(KD)]
    row0 = w * CHUNK
    b1, b2, b3 = c1 - row0, c2 - row0, c3 - row0
    zero = jnp.zeros((LANES,), jnp.float32)

    s0 = _seg_of(row0, c1, c2, c3)
    cden = zero
    cnum = [zero] * KD
    for wp in range(NW - 1):
        segl = _seg_of(wp * CHUNK + CHUNK - 1, c1, c2, c3)
        take = (wp < w) & (segl == s0)
        for k in range(KD):
            v = ta_v[pl.ds(wp * TAIL_W + LANES * k, LANES)]
            cnum[k] = cnum[k] + jnp.where(take, v, zero)
        vd = ta_v[pl.ds(wp * TAIL_W + D, LANES)]
        cden = cden + jnp.where(take, vd, zero)

    def p(g, carry):
        den = carry[0]
        nums = carry[1:]
        for j in range(GROUP):
            rg = g * GROUP + j
            off = rg * D
            den, nums, _, _ = _row_update(ctx_v, th, off, rg,
                                          b1, b2, b3, den, nums)
            inv = 1.0 / den
            for k in range(KD):
                ctx_v[pl.ds(off + LANES * k, LANES)] = nums[k] * inv
        return (den,) + nums

    lax.fori_loop(0, CHUNK // GROUP, p, (cden,) + tuple(cnum))
    pltpu.sync_copy(ctx_v, out_hbm.at[pl.ds(w * CHUNK_E, CHUNK_E)])


_l1 = pl.kernel(
    _l1_body,
    out_type=jax.ShapeDtypeStruct((NW * TAIL_W,), jnp.float32),
    mesh=_mesh,
    compiler_params=_cparams,
    scratch_types=[
        pltpu.VMEM((CHUNK_E,), jnp.float32),
        pltpu.VMEM((D,), jnp.float32),
        pltpu.VMEM((LANES,), jnp.int32),
        pltpu.VMEM((TAIL_W,), jnp.float32),
    ],
)

_l2 = pl.kernel(
    _l2_body,
    out_type=jax.ShapeDtypeStruct((T * D,), jnp.float32),
    mesh=_mesh,
    compiler_params=_cparams,
    scratch_types=[
        pltpu.VMEM((CHUNK_E,), jnp.float32),
        pltpu.VMEM((D,), jnp.float32),
        pltpu.VMEM((LANES,), jnp.int32),
        pltpu.VMEM((NW * TAIL_W,), jnp.float32),
    ],
)


@jax.jit
def kernel(context, cu_seqlens, context_theta):
    ctx_flat = context.reshape(-1)
    th_flat = context_theta.reshape(-1)
    cu_pad = jnp.concatenate(
        [cu_seqlens.astype(jnp.int32),
         jnp.zeros((LANES - cu_seqlens.shape[0],), jnp.int32)])
    tails = _l1(ctx_flat, cu_pad, th_flat)
    out_flat = _l2(ctx_flat, cu_pad, th_flat, tails)
    return out_flat.reshape(T, D)


# speedup vs baseline: 1.1006x; 1.1006x over previous
import jax
import jax.numpy as jnp
from jax import lax
from jax.experimental import pallas as pl
from jax.experimental.pallas import tpu as pltpu, tpu_sc as plsc

T = 4096
D = 128
LANES = 16
NC = 2
NS = 16
NW = NC * NS
CHUNK = T // NW
CHUNK_E = CHUNK * D
KD = D // LANES
GROUP = 16
TAIL_W = D + LANES
NSLOT = NW + NS

_mesh = plsc.VectorSubcoreMesh(core_axis_name="c", subcore_axis_name="s")
_cparams = pltpu.CompilerParams(needs_layout_passes=False)


def _cu_scalars(cu_vec):
    lane = lax.iota(jnp.int32, LANES)
    cu_f = cu_vec.astype(jnp.float32)
    c1 = jnp.sum(jnp.where(lane == 1, cu_f, 0.0)).astype(jnp.int32)
    c2 = jnp.sum(jnp.where(lane == 2, cu_f, 0.0)).astype(jnp.int32)
    c3 = jnp.sum(jnp.where(lane == 3, cu_f, 0.0)).astype(jnp.int32)
    return c1, c2, c3


def _seg_of(p, c1, c2, c3):
    return ((p >= c1).astype(jnp.int32) + (p >= c2).astype(jnp.int32)
            + (p >= c3).astype(jnp.int32))


def _tail_pass(ctx_v, th, b1, b2, b3, e_v):
    zero = jnp.zeros((LANES,), jnp.float32)

    def p(g, carry):
        den = carry[0]
        nums = carry[1:]
        for j in range(GROUP):
            rg = g * GROUP + j
            off = rg * D
            cks = [ctx_v[pl.ds(off + LANES * k, LANES)] for k in range(KD)]
            acc = cks[0] * th[0]
            for k in range(1, KD):
                acc = acc + cks[k] * th[k]
            e = jnp.exp(jnp.full((LANES,), jnp.sum(acc), jnp.float32))
            if e_v is not None:
                e_v[pl.ds(rg * LANES, LANES)] = e
            is_start = (rg == b1) | (rg == b2) | (rg == b3)
            kv = jnp.full((LANES,), jnp.where(is_start, 0.0, 1.0), jnp.float32)
            den = den * kv + e
            nums = tuple(n * kv + e * ck for n, ck in zip(nums, cks))
        return (den,) + nums

    return lax.fori_loop(0, CHUNK // GROUP, p, (zero,) * (KD + 1))


def _body(ctx_hbm, cu_hbm, th_hbm, out_hbm, tails_hbm,
          ctx_v, ctx2_v, e_v, th_v, cu_v, tl_v, ta_v):
    c = lax.axis_index("c")
    s = lax.axis_index("s")
    w = c * NS + s
    pltpu.sync_copy(ctx_hbm.at[pl.ds(w * CHUNK_E, CHUNK_E)], ctx_v)
    pltpu.sync_copy(th_hbm, th_v)
    pltpu.sync_copy(cu_hbm, cu_v)
    c1, c2, c3 = _cu_scalars(cu_v[:])
    th = [th_v[pl.ds(LANES * k, LANES)] for k in range(KD)]
    row0 = w * CHUNK
    zero = jnp.zeros((LANES,), jnp.float32)

    res = _tail_pass(ctx_v, th, c1 - row0, c2 - row0, c3 - row0, e_v)
    for k in range(KD):
        tl_v[pl.ds(LANES * k, LANES)] = res[1 + k]
    tl_v[pl.ds(D, LANES)] = res[0]
    pltpu.sync_copy(tl_v, tails_hbm.at[pl.ds(w * TAIL_W, TAIL_W)])

    @pl.when(c == 1)
    def _():
        m = s
        pltpu.sync_copy(ctx_hbm.at[pl.ds(m * CHUNK_E, CHUNK_E)], ctx2_v)
        mrow0 = m * CHUNK
        mres = _tail_pass(ctx2_v, th, c1 - mrow0, c2 - mrow0, c3 - mrow0,
                          None)
        for k in range(KD):
            tl_v[pl.ds(LANES * k, LANES)] = mres[1 + k]
        tl_v[pl.ds(D, LANES)] = mres[0]
        pltpu.sync_copy(tl_v, tails_hbm.at[pl.ds((NW + m) * TAIL_W, TAIL_W)])

    plsc.subcore_barrier()
    pltpu.sync_copy(tails_hbm, ta_v)

    s0 = _seg_of(row0, c1, c2, c3)
    cden = zero
    cnum = [zero] * KD
    for wp in range(NW - 1):
        segl = _seg_of(wp * CHUNK + CHUNK - 1, c1, c2, c3)
        take = (wp < w) & (segl == s0)
        if wp < NS:
            slot = wp + c * NW
        else:
            slot = wp
        base = slot * TAIL_W
        for k in range(KD):
            v = ta_v[pl.ds(base + LANES * k, LANES)]
            cnum[k] = cnum[k] + jnp.where(take, v, zero)
        vd = ta_v[pl.ds(base + D, LANES)]
        cden = cden + jnp.where(take, vd, zero)

    b1, b2, b3 = c1 - row0, c2 - row0, c3 - row0

    def pB(g, carry):
        den = carry[0]
        nums = carry[1:]
        for j in range(GROUP):
            rg = g * GROUP + j
            off = rg * D
            e = e_v[pl.ds(rg * LANES, LANES)]
            cks = [ctx_v[pl.ds(off + LANES * k, LANES)] for k in range(KD)]
            is_start = (rg == b1) | (rg == b2) | (rg == b3)
            kv = jnp.full((LANES,), jnp.where(is_start, 0.0, 1.0), jnp.float32)
            den = den * kv + e
            nums = tuple(n * kv + e * ck for n, ck in zip(nums, cks))
            inv = 1.0 / den
            for k in range(KD):
                ctx_v[pl.ds(off + LANES * k, LANES)] = nums[k] * inv
        return (den,) + nums

    lax.fori_loop(0, CHUNK // GROUP, pB, (cden,) + tuple(cnum))
    pltpu.sync_copy(ctx_v, out_hbm.at[pl.ds(w * CHUNK_E, CHUNK_E)])


_run = pl.kernel(
    _body,
    out_type=(jax.ShapeDtypeStruct((T * D,), jnp.float32),
              jax.ShapeDtypeStruct((NSLOT * TAIL_W,), jnp.float32)),
    mesh=_mesh,
    compiler_params=_cparams,
    scratch_types=[
        pltpu.VMEM((CHUNK_E,), jnp.float32),
        pltpu.VMEM((CHUNK_E,), jnp.float32),
        pltpu.VMEM((CHUNK * LANES,), jnp.float32),
        pltpu.VMEM((D,), jnp.float32),
        pltpu.VMEM((LANES,), jnp.int32),
        pltpu.VMEM((TAIL_W,), jnp.float32),
        pltpu.VMEM((NSLOT * TAIL_W,), jnp.float32),
    ],
)


@jax.jit
def kernel(context, cu_seqlens, context_theta):
    ctx_flat = context.reshape(-1)
    th_flat = context_theta.reshape(-1)
    cu_pad = jnp.concatenate(
        [cu_seqlens.astype(jnp.int32),
         jnp.zeros((LANES - cu_seqlens.shape[0],), jnp.int32)])
    out_flat, _ = _run(ctx_flat, cu_pad, th_flat)
    return out_flat.reshape(T, D)


# speedup vs baseline: 1.1963x; 1.0869x over previous
import jax
import jax.numpy as jnp
from jax import lax
from jax.experimental import pallas as pl
from jax.experimental.pallas import tpu as pltpu, tpu_sc as plsc

T = 4096
D = 128
LANES = 16
NS = 16
CHUNK = T // NS
CHUNK_E = CHUNK * D
KD = D // LANES
GROUP = 16
TAIL_W = D + LANES

_mesh = plsc.VectorSubcoreMesh(core_axis_name="c", subcore_axis_name="s",
                               num_cores=1)
_cparams = pltpu.CompilerParams(needs_layout_passes=False)


def _cu_scalars(cu_vec):
    lane = lax.iota(jnp.int32, LANES)
    cu_f = cu_vec.astype(jnp.float32)
    c1 = jnp.sum(jnp.where(lane == 1, cu_f, 0.0)).astype(jnp.int32)
    c2 = jnp.sum(jnp.where(lane == 2, cu_f, 0.0)).astype(jnp.int32)
    c3 = jnp.sum(jnp.where(lane == 3, cu_f, 0.0)).astype(jnp.int32)
    return c1, c2, c3


def _seg_of(p, c1, c2, c3):
    return ((p >= c1).astype(jnp.int32) + (p >= c2).astype(jnp.int32)
            + (p >= c3).astype(jnp.int32))


def _body(ctx_hbm, cu_hbm, th_hbm, out_hbm, tails_hbm,
          ctx_v, e_v, th_v, cu_v, tl_v, ta_v):
    w = lax.axis_index("s")
    pltpu.sync_copy(ctx_hbm.at[pl.ds(w * CHUNK_E, CHUNK_E)], ctx_v)
    pltpu.sync_copy(th_hbm, th_v)
    pltpu.sync_copy(cu_hbm, cu_v)
    c1, c2, c3 = _cu_scalars(cu_v[:])
    th = [th_v[pl.ds(LANES * k, LANES)] for k in range(KD)]
    row0 = w * CHUNK
    b1, b2, b3 = c1 - row0, c2 - row0, c3 - row0
    zero = jnp.zeros((LANES,), jnp.float32)

    def pA(g, carry):
        den = carry[0]
        nums = carry[1:]
        for j in range(GROUP):
            rg = g * GROUP + j
            off = rg * D
            cks = [ctx_v[pl.ds(off + LANES * k, LANES)] for k in range(KD)]
            acc = cks[0] * th[0]
            for k in range(1, KD):
                acc = acc + cks[k] * th[k]
            e = jnp.exp(jnp.full((LANES,), jnp.sum(acc), jnp.float32))
            e_v[pl.ds(rg * LANES, LANES)] = e
            is_start = (rg == b1) | (rg == b2) | (rg == b3)
            kv = jnp.full((LANES,), jnp.where(is_start, 0.0, 1.0), jnp.float32)
            den = den * kv + e
            nums = tuple(n * kv + e * ck for n, ck in zip(nums, cks))
        return (den,) + nums

    res = lax.fori_loop(0, CHUNK // GROUP, pA, (zero,) * (KD + 1))
    for k in range(KD):
        tl_v[pl.ds(LANES * k, LANES)] = res[1 + k]
    tl_v[pl.ds(D, LANES)] = res[0]
    pltpu.sync_copy(tl_v, tails_hbm.at[pl.ds(w * TAIL_W, TAIL_W)])
    plsc.subcore_barrier()
    pltpu.sync_copy(tails_hbm, ta_v)

    s0 = _seg_of(row0, c1, c2, c3)
    cden = zero
    cnum = [zero] * KD
    for wp in range(NS - 1):
        segl = _seg_of(wp * CHUNK + CHUNK - 1, c1, c2, c3)
        take = (wp < w) & (segl == s0)
        base = wp * TAIL_W
        for k in range(KD):
            v = ta_v[pl.ds(base + LANES * k, LANES)]
            cnum[k] = cnum[k] + jnp.where(take, v, zero)
        vd = ta_v[pl.ds(base + D, LANES)]
        cden = cden + jnp.where(take, vd, zero)

    def pB(g, carry):
        den = carry[0]
        nums = carry[1:]
        for j in range(GROUP):
            rg = g * GROUP + j
            off = rg * D
            e = e_v[pl.ds(rg * LANES, LANES)]
            cks = [ctx_v[pl.ds(off + LANES * k, LANES)] for k in range(KD)]
            is_start = (rg == b1) | (rg == b2) | (rg == b3)
            kv = jnp.full((LANES,), jnp.where(is_start, 0.0, 1.0), jnp.float32)
            den = den * kv + e
            nums = tuple(n * kv + e * ck for n, ck in zip(nums, cks))
            inv = 1.0 / den
            for k in range(KD):
                ctx_v[pl.ds(off + LANES * k, LANES)] = nums[k] * inv
        return (den,) + nums

    lax.fori_loop(0, CHUNK // GROUP, pB, (cden,) + tuple(cnum))
    pltpu.sync_copy(ctx_v, out_hbm.at[pl.ds(w * CHUNK_E, CHUNK_E)])


_run = pl.kernel(
    _body,
    out_type=(jax.ShapeDtypeStruct((T * D,), jnp.float32),
              jax.ShapeDtypeStruct((NS * TAIL_W,), jnp.float32)),
    mesh=_mesh,
    compiler_params=_cparams,
    scratch_types=[
        pltpu.VMEM((CHUNK_E,), jnp.float32),
        pltpu.VMEM((CHUNK * LANES,), jnp.float32),
        pltpu.VMEM((D,), jnp.float32),
        pltpu.VMEM((LANES,), jnp.int32),
        pltpu.VMEM((TAIL_W,), jnp.float32),
        pltpu.VMEM((NS * TAIL_W,), jnp.float32),
    ],
)


@jax.jit
def kernel(context, cu_seqlens, context_theta):
    ctx_flat = context.reshape(-1)
    th_flat = context_theta.reshape(-1)
    cu_pad = jnp.concatenate(
        [cu_seqlens.astype(jnp.int32),
         jnp.zeros((LANES - cu_seqlens.shape[0],), jnp.int32)])
    out_flat, _ = _run(ctx_flat, cu_pad, th_flat)
    return out_flat.reshape(T, D)
